# Initial kernel scaffold; baseline (speedup 1.0000x reference)
import jax
import jax.numpy as jnp
from jax.experimental import pallas as pl


def kernel(x, gate_w, w1, w2, w3):
    raise NotImplementedError("write your pallas kernel here")


# speedup vs baseline: 2.6263x; 2.6263x over previous
import functools

import jax
import jax.numpy as jnp
from jax import lax
from jax.experimental import pallas as pl
from jax.experimental.pallas import tpu as pltpu
from jax.experimental.pallas import tpu_sc as plsc

_NC = 2
_NS = 16
_NW = _NC * _NS
_LANES = 16


def _gate_body(x_ref, gw_ref, eidx_ref, w_ref):
    x = x_ref[...]
    s = lax.dot_general(x, gw_ref[...], (((1,), (1,)), ((), ())),
                        preferred_element_type=jnp.float32)
    lane = lax.broadcasted_iota(jnp.int32, s.shape, 1)
    m1 = jnp.max(s, axis=1)
    a1 = jnp.argmax(s, axis=1).astype(jnp.int32)
    s2 = jnp.where(lane == a1[:, None], jnp.float32(-1e30), s)
    m2 = jnp.max(s2, axis=1)
    a2 = jnp.argmax(s2, axis=1).astype(jnp.int32)
    p1 = jax.nn.sigmoid(m1 - m2)
    eidx_ref[...] = jnp.stack([a1, a2], axis=1)
    w_ref[...] = jnp.stack([p1, 1.0 - p1], axis=1)


def _gate(xf, gate_w):
    s_tot, d = xf.shape
    e = gate_w.shape[0]
    bt = 256
    return pl.pallas_call(
        _gate_body,
        grid=(s_tot // bt,),
        in_specs=[
            pl.BlockSpec((bt, d), lambda b: (b, 0)),
            pl.BlockSpec((e, d), lambda b: (0, 0)),
        ],
        out_specs=[
            pl.BlockSpec((bt, 2), lambda b: (b, 0)),
            pl.BlockSpec((bt, 2), lambda b: (b, 0)),
        ],
        out_shape=[
            jax.ShapeDtypeStruct((s_tot, 2), jnp.int32),
            jax.ShapeDtypeStruct((s_tot, 2), jnp.float32),
        ],
    )(xf, gate_w)


def _sc_gather(xf, order, pt):
    d = xf.shape[1]
    ch = 32
    rows_per_w = pt // _NW
    nch = rows_per_w // ch
    mesh = plsc.VectorSubcoreMesh(core_axis_name="c", subcore_axis_name="s")

    @functools.partial(
        pl.kernel, mesh=mesh,
        out_type=jax.ShapeDtypeStruct((pt, d), jnp.float32),
        scratch_types=[
            pltpu.VMEM((ch,), jnp.int32),
            pltpu.VMEM((ch, d), jnp.float32),
            pltpu.SemaphoreType.DMA,
        ],
    )
    def gk(x_hbm, ord_hbm, out_hbm, idx_v, rows_v, sem):
        wid = lax.axis_index("s") * _NC + lax.axis_index("c")

        def body(i, carry):
            base = wid * rows_per_w + i * ch
            pltpu.sync_copy(ord_hbm.at[pl.ds(base, ch)], idx_v)
            pltpu.async_copy(x_hbm.at[idx_v], rows_v, sem).wait()
            pltpu.sync_copy(rows_v, out_hbm.at[pl.ds(base, ch)])
            return carry

        lax.fori_loop(0, nch, body, 0)

    return gk(xf, order)


def _ffn_body(b2e_ref, xs_ref, w1_ref, w3_ref, w2_ref, ws_ref, out_ref):
    xs = xs_ref[...]
    a = lax.dot_general(xs, w1_ref[0], (((1,), (1,)), ((), ())),
                        preferred_element_type=jnp.float32)
    c = lax.dot_general(xs, w3_ref[0], (((1,), (1,)), ((), ())),
                        preferred_element_type=jnp.float32)
    h = a * jax.nn.sigmoid(a) * c
    y = lax.dot_general(h, w2_ref[0], (((1,), (1,)), ((), ())),
                        preferred_element_type=jnp.float32)
    out_ref[...] = y * ws_ref[0, 0, :][:, None]


def _ffn(b2e, xsorted, w1, w3, w2, wsort3, tb):
    pt, d = xsorted.shape
    e, f, _ = w1.shape
    nb = pt // tb
    grid_spec = pltpu.PrefetchScalarGridSpec(
        num_scalar_prefetch=1,
        grid=(nb,),
        in_specs=[
            pl.BlockSpec((tb, d), lambda b, b2e_ref: (b, 0)),
            pl.BlockSpec((1, f, d), lambda b, b2e_ref: (b2e_ref[b], 0, 0)),
            pl.BlockSpec((1, f, d), lambda b, b2e_ref: (b2e_ref[b], 0, 0)),
            pl.BlockSpec((1, d, f), lambda b, b2e_ref: (b2e_ref[b], 0, 0)),
            pl.BlockSpec((1, 1, tb), lambda b, b2e_ref: (b, 0, 0)),
        ],
        out_specs=pl.BlockSpec((tb, d), lambda b, b2e_ref: (b, 0)),
    )
    return pl.pallas_call(
        _ffn_body,
        grid_spec=grid_spec,
        out_shape=jax.ShapeDtypeStruct((pt, d), jnp.float32),
        compiler_params=pltpu.CompilerParams(
            dimension_semantics=("arbitrary",)),
    )(b2e, xsorted, w1, w3, w2, wsort3)


def _sc_combine(ysorted, dest, s_tot):
    d = ysorted.shape[1]
    ct = 16
    tok_per_w = s_tot // _NW
    nch = tok_per_w // ct
    mesh = plsc.VectorSubcoreMesh(core_axis_name="c", subcore_axis_name="s")

    @functools.partial(
        pl.kernel, mesh=mesh,
        out_type=jax.ShapeDtypeStruct((s_tot, d), jnp.float32),
        scratch_types=[
            pltpu.VMEM((2 * ct,), jnp.int32),
            pltpu.VMEM((2 * ct, d), jnp.float32),
            pltpu.VMEM((ct, d), jnp.float32),
            pltpu.SemaphoreType.DMA,
        ],
    )
    def ck(y_hbm, dest_hbm, out_hbm, idx_v, rows_v, ob_v, sem):
        wid = lax.axis_index("s") * _NC + lax.axis_index("c")

        def chunk(i, carry):
            t0 = wid * tok_per_w + i * ct
            pltpu.sync_copy(dest_hbm.at[pl.ds(2 * t0, 2 * ct)], idx_v)
            pltpu.async_copy(y_hbm.at[idx_v], rows_v, sem).wait()

            def tok(u, c2):
                for g in range(d // _LANES):
                    sl = pl.ds(g * _LANES, _LANES)
                    ob_v[u, sl] = rows_v[2 * u, sl] + rows_v[2 * u + 1, sl]
                return c2

            lax.fori_loop(0, ct, tok, 0)
            pltpu.sync_copy(ob_v, out_hbm.at[pl.ds(t0, ct)])
            return carry

        lax.fori_loop(0, nch, chunk, 0)

    return ck(ysorted, dest)


def kernel(x, gate_w, w1, w2, w3):
    b, s, d = x.shape
    e, f, _ = w1.shape
    s_tot = b * s
    k = 2
    tb = 64
    pt = s_tot * k + e * tb
    nb = pt // tb

    xf = x.reshape(s_tot, d)
    eidx, wts = _gate(xf, gate_w)

    n_flat = s_tot * k
    eflat = eidx.reshape(n_flat)
    wflat = wts.reshape(n_flat)
    perm = jnp.argsort(eflat, stable=True)
    sorted_e = eflat[perm]
    counts = jnp.zeros((e,), jnp.int32).at[eflat].add(1)
    raw_off = jnp.cumsum(counts) - counts
    rank = jnp.arange(n_flat, dtype=jnp.int32) - raw_off[sorted_e]
    cpad = ((counts + tb - 1) // tb) * tb
    pad_end = jnp.cumsum(cpad)
    pad_off = pad_end - cpad
    pos = pad_off[sorted_e] + rank
    order = jnp.zeros((pt,), jnp.int32).at[pos].set(
        (perm // k).astype(jnp.int32))
    wsort = jnp.zeros((pt,), jnp.float32).at[pos].set(wflat[perm])
    dest = jnp.zeros((n_flat,), jnp.int32).at[perm].set(pos)
    b2e = jnp.searchsorted(
        pad_end, jnp.arange(nb, dtype=jnp.int32) * tb, side="right")
    b2e = jnp.minimum(b2e, e - 1).astype(jnp.int32)
    wsort3 = wsort.reshape(nb, 1, tb)

    xsorted = _sc_gather(xf, order, pt)
    ysorted = _ffn(b2e, xsorted, w1, w3, w2, wsort3, tb)
    out = _sc_combine(ysorted, dest, s_tot)
    return out.reshape(b, s, d)


# speedup vs baseline: 2.8124x; 1.0709x over previous
import functools

import jax
import jax.numpy as jnp
from jax import lax
from jax.experimental import pallas as pl
from jax.experimental.pallas import tpu as pltpu
from jax.experimental.pallas import tpu_sc as plsc

_NC = 2
_NS = 16
_NW = _NC * _NS
_LANES = 16


def _gate_body(x_ref, gw_ref, eidx_ref, w_ref):
    x = x_ref[...]
    s = lax.dot_general(x, gw_ref[...], (((1,), (1,)), ((), ())),
                        preferred_element_type=jnp.float32)
    lane = lax.broadcasted_iota(jnp.int32, s.shape, 1)
    m1 = jnp.max(s, axis=1)
    a1 = jnp.argmax(s, axis=1).astype(jnp.int32)
    s2 = jnp.where(lane == a1[:, None], jnp.float32(-1e30), s)
    m2 = jnp.max(s2, axis=1)
    a2 = jnp.argmax(s2, axis=1).astype(jnp.int32)
    p1 = jax.nn.sigmoid(m1 - m2)
    eidx_ref[...] = jnp.stack([a1, a2], axis=1)
    w_ref[...] = jnp.stack([p1, 1.0 - p1], axis=1)


def _gate(xf, gate_w):
    s_tot, d = xf.shape
    e = gate_w.shape[0]
    bt = 256
    return pl.pallas_call(
        _gate_body,
        grid=(s_tot // bt,),
        in_specs=[
            pl.BlockSpec((bt, d), lambda b: (b, 0)),
            pl.BlockSpec((e, d), lambda b: (0, 0)),
        ],
        out_specs=[
            pl.BlockSpec((bt, 2), lambda b: (b, 0)),
            pl.BlockSpec((bt, 2), lambda b: (b, 0)),
        ],
        out_shape=[
            jax.ShapeDtypeStruct((s_tot, 2), jnp.int32),
            jax.ShapeDtypeStruct((s_tot, 2), jnp.float32),
        ],
    )(xf, gate_w)


def _sc_gather(xf, order, pt):
    d = xf.shape[1]
    ch = 32
    ring = 3
    rows_per_w = pt // _NW
    nch = rows_per_w // ch
    mesh = plsc.VectorSubcoreMesh(core_axis_name="c", subcore_axis_name="s")

    @functools.partial(
        pl.kernel, mesh=mesh,
        out_type=jax.ShapeDtypeStruct((pt, d), jnp.float32),
        scratch_types=[
            pltpu.VMEM((rows_per_w,), jnp.int32),
            pltpu.VMEM((ring, ch, d), jnp.float32),
            pltpu.SemaphoreType.DMA,
            pltpu.SemaphoreType.DMA,
            pltpu.SemaphoreType.DMA,
            pltpu.SemaphoreType.DMA,
            pltpu.SemaphoreType.DMA,
            pltpu.SemaphoreType.DMA,
        ],
    )
    def gk(x_hbm, ord_hbm, out_hbm, idx_v, rows_v, g0, g1, g2, o0, o1, o2):
        gsem = (g0, g1, g2)
        osem = (o0, o1, o2)
        wid = lax.axis_index("s") * _NC + lax.axis_index("c")
        base = wid * rows_per_w
        pltpu.sync_copy(ord_hbm.at[pl.ds(base, rows_per_w)], idx_v)

        def start_gather(i):
            b = i % ring
            return pltpu.async_copy(
                x_hbm.at[idx_v.at[pl.ds(i * ch, ch)]], rows_v.at[b], gsem[b])

        def start_out(i):
            b = i % ring
            return pltpu.async_copy(
                rows_v.at[b], out_hbm.at[pl.ds(base + i * ch, ch)], osem[b])

        g = [None] * nch
        o = [None] * nch
        for i in range(nch):
            if i >= ring:
                o[i - ring].wait()
            g[i] = start_gather(i)
            j = i - (ring - 1)
            if j >= 0:
                g[j].wait()
                o[j] = start_out(j)
        for j in range(max(0, nch - (ring - 1)), nch):
            g[j].wait()
            o[j] = start_out(j)
        for j in range(max(0, nch - ring), nch):
            o[j].wait()

    return gk(xf, order)


def _ffn_body(b2e_ref, act_ref, xs_ref, w1_ref, w3_ref, w2_ref, ws_ref,
              out_ref):
    @pl.when(act_ref[pl.program_id(0)] != 0)
    def _():
        xs = xs_ref[...]
        a = lax.dot_general(xs, w1_ref[0], (((1,), (1,)), ((), ())),
                            preferred_element_type=jnp.float32)
        c = lax.dot_general(xs, w3_ref[0], (((1,), (1,)), ((), ())),
                            preferred_element_type=jnp.float32)
        h = a * jax.nn.sigmoid(a) * c
        y = lax.dot_general(h, w2_ref[0], (((1,), (1,)), ((), ())),
                            preferred_element_type=jnp.float32)
        out_ref[...] = y * ws_ref[0, 0, :][:, None]


def _ffn(b2e, act, xsorted, w1, w3, w2, wsort3, tb):
    pt, d = xsorted.shape
    e, f, _ = w1.shape
    nb = pt // tb
    grid_spec = pltpu.PrefetchScalarGridSpec(
        num_scalar_prefetch=2,
        grid=(nb,),
        in_specs=[
            pl.BlockSpec((tb, d), lambda b, b2e_ref, act_ref: (b, 0)),
            pl.BlockSpec((1, f, d),
                         lambda b, b2e_ref, act_ref: (b2e_ref[b], 0, 0)),
            pl.BlockSpec((1, f, d),
                         lambda b, b2e_ref, act_ref: (b2e_ref[b], 0, 0)),
            pl.BlockSpec((1, d, f),
                         lambda b, b2e_ref, act_ref: (b2e_ref[b], 0, 0)),
            pl.BlockSpec((1, 1, tb),
                         lambda b, b2e_ref, act_ref: (b, 0, 0)),
        ],
        out_specs=pl.BlockSpec((tb, d), lambda b, b2e_ref, act_ref: (b, 0)),
    )
    return pl.pallas_call(
        _ffn_body,
        grid_spec=grid_spec,
        out_shape=jax.ShapeDtypeStruct((pt, d), jnp.float32),
        compiler_params=pltpu.CompilerParams(
            dimension_semantics=("arbitrary",)),
    )(b2e, act, xsorted, w1, w3, w2, wsort3)


def _sc_combine(ysorted, dest, s_tot):
    d = ysorted.shape[1]
    ct = 16
    tok_per_w = s_tot // _NW
    nch = tok_per_w // ct
    mesh = plsc.VectorSubcoreMesh(core_axis_name="c", subcore_axis_name="s")

    @functools.partial(
        pl.kernel, mesh=mesh,
        out_type=jax.ShapeDtypeStruct((s_tot, d), jnp.float32),
        scratch_types=[
            pltpu.VMEM((2 * tok_per_w,), jnp.int32),
            pltpu.VMEM((2, 2 * ct, d), jnp.float32),
            pltpu.VMEM((2, ct, d), jnp.float32),
            pltpu.SemaphoreType.DMA,
            pltpu.SemaphoreType.DMA,
            pltpu.SemaphoreType.DMA,
            pltpu.SemaphoreType.DMA,
        ],
    )
    def ck(y_hbm, dest_hbm, out_hbm, idx_v, rows_v, ob_v, g0, g1, o0, o1):
        gsem = (g0, g1)
        osem = (o0, o1)
        wid = lax.axis_index("s") * _NC + lax.axis_index("c")
        t_base = wid * tok_per_w
        pltpu.sync_copy(dest_hbm.at[pl.ds(2 * t_base, 2 * tok_per_w)], idx_v)

        def start_gather(i):
            b = i % 2
            return pltpu.async_copy(
                y_hbm.at[idx_v.at[pl.ds(i * 2 * ct, 2 * ct)]],
                rows_v.at[b], gsem[b])

        g = [None] * nch
        o = [None] * nch
        g[0] = start_gather(0)
        for i in range(nch):
            b = i % 2
            if i + 1 < nch:
                g[i + 1] = start_gather(i + 1)
            g[i].wait()
            if i >= 2:
                o[i - 2].wait()
            rb = rows_v.at[b]
            obb = ob_v.at[b]

            def tok(u, c2):
                for gi in range(d // _LANES):
                    sl = pl.ds(gi * _LANES, _LANES)
                    obb[u, sl] = rb[2 * u, sl] + rb[2 * u + 1, sl]
                return c2

            lax.fori_loop(0, ct, tok, 0)
            o[i] = pltpu.async_copy(
                obb, out_hbm.at[pl.ds(t_base + i * ct, ct)], osem[b])
        for j in range(max(0, nch - 2), nch):
            o[j].wait()

    return ck(ysorted, dest)


def kernel(x, gate_w, w1, w2, w3):
    b, s, d = x.shape
    e, f, _ = w1.shape
    s_tot = b * s
    k = 2
    tb = 64
    pt = s_tot * k + e * tb
    nb = pt // tb

    xf = x.reshape(s_tot, d)
    eidx, wts = _gate(xf, gate_w)

    n_flat = s_tot * k
    eflat = eidx.reshape(n_flat)
    wflat = wts.reshape(n_flat)
    perm = jnp.argsort(eflat, stable=True)
    sorted_e = eflat[perm]
    counts = jnp.zeros((e,), jnp.int32).at[eflat].add(1)
    raw_off = jnp.cumsum(counts) - counts
    rank = jnp.arange(n_flat, dtype=jnp.int32) - raw_off[sorted_e]
    cpad = ((counts + tb - 1) // tb) * tb
    pad_end = jnp.cumsum(cpad)
    pad_off = pad_end - cpad
    pos = pad_off[sorted_e] + rank
    order = jnp.zeros((pt,), jnp.int32).at[pos].set(
        (perm // k).astype(jnp.int32))
    wsort = jnp.zeros((pt,), jnp.float32).at[pos].set(wflat[perm])
    dest = jnp.zeros((n_flat,), jnp.int32).at[perm].set(pos)
    block_starts = jnp.arange(nb, dtype=jnp.int32) * tb
    b2e = jnp.searchsorted(pad_end, block_starts, side="right")
    b2e = jnp.minimum(b2e, e - 1).astype(jnp.int32)
    act = (block_starts < pad_end[-1]).astype(jnp.int32)
    wsort3 = wsort.reshape(nb, 1, tb)

    xsorted = _sc_gather(xf, order, pt)
    ysorted = _ffn(b2e, act, xsorted, w1, w3, w2, wsort3, tb)
    out = _sc_combine(ysorted, dest, s_tot)
    return out.reshape(b, s, d)


# speedup vs baseline: 3.0226x; 1.0747x over previous
import functools

import jax
import jax.numpy as jnp
from jax import lax
from jax.experimental import pallas as pl
from jax.experimental.pallas import tpu as pltpu
from jax.experimental.pallas import tpu_sc as plsc

_NC = 2
_NS = 16
_NW = _NC * _NS
_LANES = 16


def _gate_body(x_ref, gw_ref, eidx_ref, w_ref):
    x = x_ref[...]
    s = lax.dot_general(x, gw_ref[...], (((1,), (1,)), ((), ())),
                        preferred_element_type=jnp.float32)
    lane = lax.broadcasted_iota(jnp.int32, s.shape, 1)
    m1 = jnp.max(s, axis=1)
    a1 = jnp.argmax(s, axis=1).astype(jnp.int32)
    s2 = jnp.where(lane == a1[:, None], jnp.float32(-1e30), s)
    m2 = jnp.max(s2, axis=1)
    a2 = jnp.argmax(s2, axis=1).astype(jnp.int32)
    p1 = jax.nn.sigmoid(m1 - m2)
    eidx_ref[...] = jnp.stack([a1, a2], axis=1)
    w_ref[...] = jnp.stack([p1, 1.0 - p1], axis=1)


def _gate(xf, gate_w):
    s_tot, d = xf.shape
    e = gate_w.shape[0]
    bt = 256
    return pl.pallas_call(
        _gate_body,
        grid=(s_tot // bt,),
        in_specs=[
            pl.BlockSpec((bt, d), lambda b: (b, 0)),
            pl.BlockSpec((e, d), lambda b: (0, 0)),
        ],
        out_specs=[
            pl.BlockSpec((bt, 2), lambda b: (b, 0)),
            pl.BlockSpec((bt, 2), lambda b: (b, 0)),
        ],
        out_shape=[
            jax.ShapeDtypeStruct((s_tot, 2), jnp.int32),
            jax.ShapeDtypeStruct((s_tot, 2), jnp.float32),
        ],
    )(xf, gate_w)


def _sc_gather(x3, order, pt):
    ch = 32
    ring = 3
    rows_per_w = pt // _NW
    nch = rows_per_w // ch
    mesh = plsc.VectorSubcoreMesh(core_axis_name="c", subcore_axis_name="s")

    @functools.partial(
        pl.kernel, mesh=mesh,
        out_type=jax.ShapeDtypeStruct((pt, 8, 128), jnp.float32),
        scratch_types=[
            pltpu.VMEM((rows_per_w,), jnp.int32),
            pltpu.VMEM((ring, ch, 8, 128), jnp.float32),
            pltpu.SemaphoreType.DMA,
            pltpu.SemaphoreType.DMA,
            pltpu.SemaphoreType.DMA,
            pltpu.SemaphoreType.DMA,
            pltpu.SemaphoreType.DMA,
            pltpu.SemaphoreType.DMA,
        ],
    )
    def gk(x_hbm, ord_hbm, out_hbm, idx_v, rows_v, g0, g1, g2, o0, o1, o2):
        gsem = (g0, g1, g2)
        osem = (o0, o1, o2)
        wid = lax.axis_index("s") * _NC + lax.axis_index("c")
        base = wid * rows_per_w
        pltpu.sync_copy(ord_hbm.at[pl.ds(base, rows_per_w)], idx_v)

        def start_gather(i):
            b = i % ring
            return pltpu.async_copy(
                x_hbm.at[idx_v.at[pl.ds(i * ch, ch)]], rows_v.at[b], gsem[b])

        def start_out(i):
            b = i % ring
            return pltpu.async_copy(
                rows_v.at[b], out_hbm.at[pl.ds(base + i * ch, ch)], osem[b])

        g = [None] * nch
        o = [None] * nch
        for i in range(nch):
            if i >= ring:
                o[i - ring].wait()
            g[i] = start_gather(i)
            j = i - (ring - 1)
            if j >= 0:
                g[j].wait()
                o[j] = start_out(j)
        for j in range(max(0, nch - (ring - 1)), nch):
            g[j].wait()
            o[j] = start_out(j)
        for j in range(max(0, nch - ring), nch):
            o[j].wait()

    return gk(x3, order)


def _ffn_body(b2e_ref, act_ref, xs_ref, w1_ref, w3_ref, w2_ref, ws_ref,
              out_ref):
    @pl.when(act_ref[pl.program_id(0)] != 0)
    def _():
        xs = xs_ref[...].reshape(xs_ref.shape[0], 1024)
        a = lax.dot_general(xs, w1_ref[0], (((1,), (1,)), ((), ())),
                            preferred_element_type=jnp.float32)
        c = lax.dot_general(xs, w3_ref[0], (((1,), (1,)), ((), ())),
                            preferred_element_type=jnp.float32)
        h = a * jax.nn.sigmoid(a) * c
        y = lax.dot_general(h, w2_ref[0], (((1,), (1,)), ((), ())),
                            preferred_element_type=jnp.float32)
        out_ref[...] = y * ws_ref[0, 0, :][:, None]


def _ffn(b2e, act, xsorted3, w1, w3, w2, wsort3, tb):
    pt = xsorted3.shape[0]
    e, f, d = w1.shape
    nb = pt // tb
    grid_spec = pltpu.PrefetchScalarGridSpec(
        num_scalar_prefetch=2,
        grid=(nb,),
        in_specs=[
            pl.BlockSpec((tb, 8, 128), lambda b, b2e_ref, act_ref: (b, 0, 0)),
            pl.BlockSpec((1, f, d),
                         lambda b, b2e_ref, act_ref: (b2e_ref[b], 0, 0)),
            pl.BlockSpec((1, f, d),
                         lambda b, b2e_ref, act_ref: (b2e_ref[b], 0, 0)),
            pl.BlockSpec((1, d, f),
                         lambda b, b2e_ref, act_ref: (b2e_ref[b], 0, 0)),
            pl.BlockSpec((1, 1, tb),
                         lambda b, b2e_ref, act_ref: (b, 0, 0)),
        ],
        out_specs=pl.BlockSpec((tb, d), lambda b, b2e_ref, act_ref: (b, 0)),
    )
    return pl.pallas_call(
        _ffn_body,
        grid_spec=grid_spec,
        out_shape=jax.ShapeDtypeStruct((pt, d), jnp.float32),
        compiler_params=pltpu.CompilerParams(
            dimension_semantics=("arbitrary",)),
    )(b2e, act, xsorted3, w1, w3, w2, wsort3)


def _sc_combine(ysorted, dest, s_tot):
    d = ysorted.shape[1]
    ct = 16
    tok_per_w = s_tot // _NW
    nch = tok_per_w // ct
    mesh = plsc.VectorSubcoreMesh(core_axis_name="c", subcore_axis_name="s")

    @functools.partial(
        pl.kernel, mesh=mesh,
        out_type=jax.ShapeDtypeStruct((s_tot, d), jnp.float32),
        scratch_types=[
            pltpu.VMEM((2 * tok_per_w,), jnp.int32),
            pltpu.VMEM((2, 2 * ct, d), jnp.float32),
            pltpu.VMEM((2, ct, d), jnp.float32),
            pltpu.SemaphoreType.DMA,
            pltpu.SemaphoreType.DMA,
            pltpu.SemaphoreType.DMA,
            pltpu.SemaphoreType.DMA,
        ],
    )
    def ck(y_hbm, dest_hbm, out_hbm, idx_v, rows_v, ob_v, g0, g1, o0, o1):
        gsem = (g0, g1)
        osem = (o0, o1)
        wid = lax.axis_index("s") * _NC + lax.axis_index("c")
        t_base = wid * tok_per_w
        pltpu.sync_copy(dest_hbm.at[pl.ds(2 * t_base, 2 * tok_per_w)], idx_v)

        def start_gather(i):
            b = i % 2
            return pltpu.async_copy(
                y_hbm.at[idx_v.at[pl.ds(i * 2 * ct, 2 * ct)]],
                rows_v.at[b], gsem[b])

        g = [None] * nch
        o = [None] * nch
        g[0] = start_gather(0)
        for i in range(nch):
            b = i % 2
            if i + 1 < nch:
                g[i + 1] = start_gather(i + 1)
            g[i].wait()
            if i >= 2:
                o[i - 2].wait()
            rb = rows_v.at[b]
            obb = ob_v.at[b]

            def tok(u, c2):
                for gi in range(d // _LANES):
                    sl = pl.ds(gi * _LANES, _LANES)
                    obb[u, sl] = rb[2 * u, sl] + rb[2 * u + 1, sl]
                return c2

            lax.fori_loop(0, ct, tok, 0)
            o[i] = pltpu.async_copy(
                obb, out_hbm.at[pl.ds(t_base + i * ct, ct)], osem[b])
        for j in range(max(0, nch - 2), nch):
            o[j].wait()

    return ck(ysorted, dest)


def kernel(x, gate_w, w1, w2, w3):
    b, s, d = x.shape
    e, f, _ = w1.shape
    s_tot = b * s
    k = 2
    tb = 64
    pt = s_tot * k + e * tb
    nb = pt // tb

    xf = x.reshape(s_tot, d)
    x3 = xf.reshape(s_tot, 8, 128)
    eidx, wts = _gate(xf, gate_w)

    n_flat = s_tot * k
    eflat = eidx.reshape(n_flat)
    wflat = wts.reshape(n_flat)
    i_arange = jnp.arange(n_flat, dtype=jnp.int32)
    key = eflat * n_flat + i_arange
    skey, w_sorted = lax.sort((key, wflat), num_keys=1)
    sorted_e = skey // n_flat
    perm = skey % n_flat
    boundary = jnp.concatenate(
        [jnp.ones((1,), jnp.int32),
         (sorted_e[1:] != sorted_e[:-1]).astype(jnp.int32)])
    seg_start = lax.cummax(jnp.where(boundary == 1, i_arange, 0))
    prev_start = jnp.concatenate([jnp.zeros((1,), jnp.int32),
                                  seg_start[:-1]])
    prev_len = i_arange - prev_start
    pad_amt = jnp.where((boundary == 1) & (i_arange > 0),
                        (-prev_len) % tb, 0)
    pos = i_arange + jnp.cumsum(pad_amt)
    order = jnp.zeros((pt,), jnp.int32).at[pos].set(
        (perm // k).astype(jnp.int32))
    wsort = jnp.zeros((pt,), jnp.float32).at[pos].set(w_sorted)
    _, dest = lax.sort((perm, pos), num_keys=1)
    blk = pos // tb
    b2e = lax.cummax(jnp.zeros((nb,), jnp.int32).at[blk].max(sorted_e))
    act = (jnp.arange(nb, dtype=jnp.int32) <= blk[-1]).astype(jnp.int32)
    wsort3 = wsort.reshape(nb, 1, tb)

    xsorted3 = _sc_gather(x3, order, pt)
    ysorted = _ffn(b2e, act, xsorted3, w1, w3, w2, wsort3, tb)
    out = _sc_combine(ysorted, dest, s_tot)
    return out.reshape(b, s, d)


# speedup vs baseline: 4.1405x; 1.3699x over previous
import functools

import jax
import jax.numpy as jnp
from jax import lax
from jax.experimental import pallas as pl
from jax.experimental.pallas import tpu as pltpu
from jax.experimental.pallas import tpu_sc as plsc

_NC = 2
_NS = 16
_NW = _NC * _NS
_LANES = 16


def _gate_body(x_ref, gw_ref, eidx_ref, w_ref):
    x = x_ref[...]
    s = lax.dot_general(x, gw_ref[...], (((1,), (1,)), ((), ())),
                        preferred_element_type=jnp.float32)
    lane = lax.broadcasted_iota(jnp.int32, s.shape, 1)
    m1 = jnp.max(s, axis=1)
    a1 = jnp.argmax(s, axis=1).astype(jnp.int32)
    s2 = jnp.where(lane == a1[:, None], jnp.float32(-1e30), s)
    m2 = jnp.max(s2, axis=1)
    a2 = jnp.argmax(s2, axis=1).astype(jnp.int32)
    p1 = jax.nn.sigmoid(m1 - m2)
    eidx_ref[...] = jnp.stack([a1, a2], axis=1)
    w_ref[...] = jnp.stack([p1, 1.0 - p1], axis=1)


def _gate(xf, gate_w):
    s_tot, d = xf.shape
    e = gate_w.shape[0]
    bt = 256
    return pl.pallas_call(
        _gate_body,
        grid=(s_tot // bt,),
        in_specs=[
            pl.BlockSpec((bt, d), lambda b: (b, 0)),
            pl.BlockSpec((e, d), lambda b: (0, 0)),
        ],
        out_specs=[
            pl.BlockSpec((bt, 2), lambda b: (b, 0)),
            pl.BlockSpec((bt, 2), lambda b: (b, 0)),
        ],
        out_shape=[
            jax.ShapeDtypeStruct((s_tot, 2), jnp.int32),
            jax.ShapeDtypeStruct((s_tot, 2), jnp.float32),
        ],
    )(xf, gate_w)


def _sc_gather(x3, order, pt):
    ch = 32
    ring = 3
    rows_per_w = pt // _NW
    nch = rows_per_w // ch
    mesh = plsc.VectorSubcoreMesh(core_axis_name="c", subcore_axis_name="s")

    @functools.partial(
        pl.kernel, mesh=mesh,
        out_type=jax.ShapeDtypeStruct((pt, 8, 128), jnp.float32),
        scratch_types=[
            pltpu.VMEM((rows_per_w,), jnp.int32),
            pltpu.VMEM((ring, ch, 8, 128), jnp.float32),
            pltpu.SemaphoreType.DMA,
            pltpu.SemaphoreType.DMA,
            pltpu.SemaphoreType.DMA,
            pltpu.SemaphoreType.DMA,
            pltpu.SemaphoreType.DMA,
            pltpu.SemaphoreType.DMA,
        ],
    )
    def gk(x_hbm, ord_hbm, out_hbm, idx_v, rows_v, g0, g1, g2, o0, o1, o2):
        gsem = (g0, g1, g2)
        osem = (o0, o1, o2)
        wid = lax.axis_index("s") * _NC + lax.axis_index("c")
        base = wid * rows_per_w
        pltpu.sync_copy(ord_hbm.at[pl.ds(base, rows_per_w)], idx_v)

        def start_gather(i):
            b = i % ring
            return pltpu.async_copy(
                x_hbm.at[idx_v.at[pl.ds(i * ch, ch)]], rows_v.at[b], gsem[b])

        def start_out(i):
            b = i % ring
            return pltpu.async_copy(
                rows_v.at[b], out_hbm.at[pl.ds(base + i * ch, ch)], osem[b])

        g = [None] * nch
        o = [None] * nch
        for i in range(nch):
            if i >= ring:
                o[i - ring].wait()
            g[i] = start_gather(i)
            j = i - (ring - 1)
            if j >= 0:
                g[j].wait()
                o[j] = start_out(j)
        for j in range(max(0, nch - (ring - 1)), nch):
            g[j].wait()
            o[j] = start_out(j)
        for j in range(max(0, nch - ring), nch):
            o[j].wait()

    return gk(x3, order)


def _ffn_body(b2e_ref, act_ref, xs_ref, w1_ref, w3_ref, w2_ref, ws_ref,
              out_ref):
    @pl.when(act_ref[pl.program_id(0)] != 0)
    def _():
        xs = xs_ref[...].reshape(xs_ref.shape[0], 1024)
        a = lax.dot_general(xs, w1_ref[0], (((1,), (1,)), ((), ())),
                            preferred_element_type=jnp.float32)
        c = lax.dot_general(xs, w3_ref[0], (((1,), (1,)), ((), ())),
                            preferred_element_type=jnp.float32)
        h = a * jax.nn.sigmoid(a) * c
        y = lax.dot_general(h, w2_ref[0], (((1,), (1,)), ((), ())),
                            preferred_element_type=jnp.float32)
        out_ref[...] = y * ws_ref[0, 0, :][:, None]


def _ffn(b2e, act, xsorted3, w1, w3, w2, wsort3, tb):
    pt = xsorted3.shape[0]
    e, f, d = w1.shape
    nb = pt // tb
    grid_spec = pltpu.PrefetchScalarGridSpec(
        num_scalar_prefetch=2,
        grid=(nb,),
        in_specs=[
            pl.BlockSpec((tb, 8, 128), lambda b, b2e_ref, act_ref: (b, 0, 0)),
            pl.BlockSpec((1, f, d),
                         lambda b, b2e_ref, act_ref: (b2e_ref[b], 0, 0)),
            pl.BlockSpec((1, f, d),
                         lambda b, b2e_ref, act_ref: (b2e_ref[b], 0, 0)),
            pl.BlockSpec((1, d, f),
                         lambda b, b2e_ref, act_ref: (b2e_ref[b], 0, 0)),
            pl.BlockSpec((1, 1, tb),
                         lambda b, b2e_ref, act_ref: (b, 0, 0)),
        ],
        out_specs=pl.BlockSpec((tb, d), lambda b, b2e_ref, act_ref: (b, 0)),
    )
    return pl.pallas_call(
        _ffn_body,
        grid_spec=grid_spec,
        out_shape=jax.ShapeDtypeStruct((pt, d), jnp.float32),
        compiler_params=pltpu.CompilerParams(
            dimension_semantics=("arbitrary",)),
    )(b2e, act, xsorted3, w1, w3, w2, wsort3)


def _sc_combine(ysorted, dest, s_tot):
    d = ysorted.shape[1]
    ct = 16
    tok_per_w = s_tot // _NW
    nch = tok_per_w // ct
    mesh = plsc.VectorSubcoreMesh(core_axis_name="c", subcore_axis_name="s")

    @functools.partial(
        pl.kernel, mesh=mesh,
        out_type=jax.ShapeDtypeStruct((s_tot, d), jnp.float32),
        scratch_types=[
            pltpu.VMEM((2 * tok_per_w,), jnp.int32),
            pltpu.VMEM((2, 2 * ct, d), jnp.float32),
            pltpu.VMEM((2, ct, d), jnp.float32),
            pltpu.SemaphoreType.DMA,
            pltpu.SemaphoreType.DMA,
            pltpu.SemaphoreType.DMA,
            pltpu.SemaphoreType.DMA,
        ],
    )
    def ck(y_hbm, dest_hbm, out_hbm, idx_v, rows_v, ob_v, g0, g1, o0, o1):
        gsem = (g0, g1)
        osem = (o0, o1)
        wid = lax.axis_index("s") * _NC + lax.axis_index("c")
        t_base = wid * tok_per_w
        pltpu.sync_copy(dest_hbm.at[pl.ds(2 * t_base, 2 * tok_per_w)], idx_v)

        def start_gather(i):
            b = i % 2
            return pltpu.async_copy(
                y_hbm.at[idx_v.at[pl.ds(i * 2 * ct, 2 * ct)]],
                rows_v.at[b], gsem[b])

        g = [None] * nch
        o = [None] * nch
        g[0] = start_gather(0)
        for i in range(nch):
            b = i % 2
            if i + 1 < nch:
                g[i + 1] = start_gather(i + 1)
            g[i].wait()
            if i >= 2:
                o[i - 2].wait()
            rb = rows_v.at[b]
            obb = ob_v.at[b]

            def tok(u, c2):
                for gi in range(d // _LANES):
                    sl = pl.ds(gi * _LANES, _LANES)
                    obb[u, sl] = rb[2 * u, sl] + rb[2 * u + 1, sl]
                return c2

            lax.fori_loop(0, ct, tok, 0)
            o[i] = pltpu.async_copy(
                obb, out_hbm.at[pl.ds(t_base + i * ct, ct)], osem[b])
        for j in range(max(0, nch - 2), nch):
            o[j].wait()

    return ck(ysorted, dest)


def kernel(x, gate_w, w1, w2, w3):
    b, s, d = x.shape
    e, f, _ = w1.shape
    s_tot = b * s
    k = 2
    tb = 64
    pt = s_tot * k + e * tb
    nb = pt // tb

    xf = x.reshape(s_tot, d)
    x3 = xf.reshape(s_tot, 8, 128)
    eidx, wts = _gate(xf, gate_w)

    n_flat = s_tot * k
    eflat = eidx.reshape(n_flat)
    wflat = wts.reshape(n_flat)
    i_arange = jnp.arange(n_flat, dtype=jnp.int32)
    key = eflat * n_flat + i_arange
    skey, w_sorted = lax.sort((key, wflat), num_keys=1)
    sorted_e = skey // n_flat
    perm = skey % n_flat
    boundary = jnp.concatenate(
        [jnp.ones((1,), jnp.int32),
         (sorted_e[1:] != sorted_e[:-1]).astype(jnp.int32)])
    seg_start = lax.cummax(jnp.where(boundary == 1, i_arange, 0))
    prev_start = jnp.concatenate([jnp.zeros((1,), jnp.int32),
                                  seg_start[:-1]])
    prev_len = i_arange - prev_start
    pad_amt = jnp.where((boundary == 1) & (i_arange > 0),
                        (-prev_len) % tb, 0)
    pos = i_arange + jnp.cumsum(pad_amt)
    order = (jnp.arange(pt, dtype=jnp.int32) % s_tot).at[pos].set(
        (perm // k).astype(jnp.int32))
    wsort = jnp.zeros((pt,), jnp.float32).at[pos].set(w_sorted)
    _, dest = lax.sort((perm, pos), num_keys=1)
    blk = pos // tb
    b2e = lax.cummax(jnp.zeros((nb,), jnp.int32).at[blk].max(sorted_e))
    act = (jnp.arange(nb, dtype=jnp.int32) <= blk[-1]).astype(jnp.int32)
    wsort3 = wsort.reshape(nb, 1, tb)

    xsorted3 = _sc_gather(x3, order, pt)
    ysorted = _ffn(b2e, act, xsorted3, w1, w3, w2, wsort3, tb)
    out = _sc_combine(ysorted, dest, s_tot)
    return out.reshape(b, s, d)


# speedup vs baseline: 4.6548x; 1.1242x over previous
import functools

import jax
import jax.numpy as jnp
from jax import lax
from jax.experimental import pallas as pl
from jax.experimental.pallas import tpu as pltpu
from jax.experimental.pallas import tpu_sc as plsc

_NC = 2
_NS = 16
_NW = _NC * _NS
_LANES = 16


def _gate_body(x_ref, gw_ref, eidx_ref, w_ref):
    x = x_ref[...]
    s = lax.dot_general(x, gw_ref[...], (((1,), (1,)), ((), ())),
                        preferred_element_type=jnp.float32)
    lane = lax.broadcasted_iota(jnp.int32, s.shape, 1)
    m1 = jnp.max(s, axis=1)
    a1 = jnp.argmax(s, axis=1).astype(jnp.int32)
    s2 = jnp.where(lane == a1[:, None], jnp.float32(-1e30), s)
    m2 = jnp.max(s2, axis=1)
    a2 = jnp.argmax(s2, axis=1).astype(jnp.int32)
    p1 = jax.nn.sigmoid(m1 - m2)
    eidx_ref[...] = jnp.stack([a1, a2], axis=1)
    w_ref[...] = jnp.stack([p1, 1.0 - p1], axis=1)


def _gate(xf, gate_w):
    s_tot, d = xf.shape
    e = gate_w.shape[0]
    bt = 256
    return pl.pallas_call(
        _gate_body,
        grid=(s_tot // bt,),
        in_specs=[
            pl.BlockSpec((bt, d), lambda b: (b, 0)),
            pl.BlockSpec((e, d), lambda b: (0, 0)),
        ],
        out_specs=[
            pl.BlockSpec((bt, 2), lambda b: (b, 0)),
            pl.BlockSpec((bt, 2), lambda b: (b, 0)),
        ],
        out_shape=[
            jax.ShapeDtypeStruct((s_tot, 2), jnp.int32),
            jax.ShapeDtypeStruct((s_tot, 2), jnp.float32),
        ],
    )(xf, gate_w)


def _sc_dispatch(x3, tok2, pos2, pt):
    nch_tot, ch = tok2.shape
    nch = nch_tot // _NW
    ring = 3
    mesh = plsc.VectorSubcoreMesh(core_axis_name="c", subcore_axis_name="s")

    @functools.partial(
        pl.kernel, mesh=mesh,
        out_type=jax.ShapeDtypeStruct((pt, 8, 128), jnp.float32),
        scratch_types=[
            pltpu.VMEM((nch, ch), jnp.int32),
            pltpu.VMEM((nch, ch), jnp.int32),
            pltpu.VMEM((ring, ch, 8, 128), jnp.float32),
            pltpu.SemaphoreType.DMA,
            pltpu.SemaphoreType.DMA,
            pltpu.SemaphoreType.DMA,
            pltpu.SemaphoreType.DMA,
            pltpu.SemaphoreType.DMA,
            pltpu.SemaphoreType.DMA,
        ],
    )
    def dk(x_hbm, tok_hbm, pos_hbm, out_hbm, tok_v, pos_v, rows_v,
           g0, g1, g2, o0, o1, o2):
        gsem = (g0, g1, g2)
        osem = (o0, o1, o2)
        wid = lax.axis_index("s") * _NC + lax.axis_index("c")
        pltpu.sync_copy(tok_hbm.at[pl.ds(wid * nch, nch)], tok_v)
        pltpu.sync_copy(pos_hbm.at[pl.ds(wid * nch, nch)], pos_v)

        def start_g(i):
            b = i % ring
            return pltpu.async_copy(
                x_hbm.at[tok_v.at[i]], rows_v.at[b], gsem[b])

        def start_o(i):
            b = i % ring
            return pltpu.async_copy(
                rows_v.at[b], out_hbm.at[pos_v.at[i]], osem[b])

        g = [None] * nch
        o = [None] * nch
        g[0] = start_g(0)
        for i in range(nch):
            if i + 1 < nch:
                if i + 1 >= ring:
                    o[i + 1 - ring].wait()
                g[i + 1] = start_g(i + 1)
            g[i].wait()
            o[i] = start_o(i)
        for j in range(max(0, nch - ring), nch):
            o[j].wait()

    return dk(x3, tok2, pos2)


def _ffn_body(b2e_ref, act_ref, xs_ref, w1_ref, w3_ref, w2_ref, out_ref):
    @pl.when(act_ref[pl.program_id(0)] != 0)
    def _():
        xs = xs_ref[...].reshape(xs_ref.shape[0], 1024)
        a = lax.dot_general(xs, w1_ref[0], (((1,), (1,)), ((), ())),
                            preferred_element_type=jnp.float32)
        c = lax.dot_general(xs, w3_ref[0], (((1,), (1,)), ((), ())),
                            preferred_element_type=jnp.float32)
        h = a * jax.nn.sigmoid(a) * c
        y = lax.dot_general(h, w2_ref[0], (((1,), (1,)), ((), ())),
                            preferred_element_type=jnp.float32)
        out_ref[...] = y


def _ffn(b2e, act, xsorted3, w1, w3, w2, tb):
    pt = xsorted3.shape[0]
    e, f, d = w1.shape
    nb = pt // tb
    grid_spec = pltpu.PrefetchScalarGridSpec(
        num_scalar_prefetch=2,
        grid=(nb,),
        in_specs=[
            pl.BlockSpec((tb, 8, 128), lambda b, b2e_ref, act_ref: (b, 0, 0)),
            pl.BlockSpec((1, f, d),
                         lambda b, b2e_ref, act_ref: (b2e_ref[b], 0, 0)),
            pl.BlockSpec((1, f, d),
                         lambda b, b2e_ref, act_ref: (b2e_ref[b], 0, 0)),
            pl.BlockSpec((1, d, f),
                         lambda b, b2e_ref, act_ref: (b2e_ref[b], 0, 0)),
        ],
        out_specs=pl.BlockSpec((tb, d), lambda b, b2e_ref, act_ref: (b, 0)),
    )
    return pl.pallas_call(
        _ffn_body,
        grid_spec=grid_spec,
        out_shape=jax.ShapeDtypeStruct((pt, d), jnp.float32),
        compiler_params=pltpu.CompilerParams(
            dimension_semantics=("arbitrary",)),
    )(b2e, act, xsorted3, w1, w3, w2)


def _sc_combine(ysorted, dest, wexp, s_tot):
    d = ysorted.shape[1]
    ct = 16
    tok_per_w = s_tot // _NW
    nch = tok_per_w // ct
    mesh = plsc.VectorSubcoreMesh(core_axis_name="c", subcore_axis_name="s")

    @functools.partial(
        pl.kernel, mesh=mesh,
        out_type=jax.ShapeDtypeStruct((s_tot, d), jnp.float32),
        scratch_types=[
            pltpu.VMEM((2 * tok_per_w,), jnp.int32),
            pltpu.VMEM((2 * tok_per_w, _LANES), jnp.float32),
            pltpu.VMEM((2, 2 * ct, d), jnp.float32),
            pltpu.VMEM((2, ct, d), jnp.float32),
            pltpu.SemaphoreType.DMA,
            pltpu.SemaphoreType.DMA,
            pltpu.SemaphoreType.DMA,
            pltpu.SemaphoreType.DMA,
        ],
    )
    def ck(y_hbm, dest_hbm, w_hbm, out_hbm, idx_v, wv, rows_v, ob_v,
           g0, g1, o0, o1):
        gsem = (g0, g1)
        osem = (o0, o1)
        wid = lax.axis_index("s") * _NC + lax.axis_index("c")
        t_base = wid * tok_per_w
        pltpu.sync_copy(dest_hbm.at[pl.ds(2 * t_base, 2 * tok_per_w)], idx_v)
        pltpu.sync_copy(w_hbm.at[pl.ds(2 * t_base, 2 * tok_per_w)], wv)

        def start_gather(i):
            b = i % 2
            return pltpu.async_copy(
                y_hbm.at[idx_v.at[pl.ds(i * 2 * ct, 2 * ct)]],
                rows_v.at[b], gsem[b])

        g = [None] * nch
        o = [None] * nch
        g[0] = start_gather(0)
        for i in range(nch):
            b = i % 2
            if i + 1 < nch:
                g[i + 1] = start_gather(i + 1)
            g[i].wait()
            if i >= 2:
                o[i - 2].wait()
            rb = rows_v.at[b]
            obb = ob_v.at[b]

            def tok(u, c2):
                w0 = wv[i * 2 * ct + 2 * u, :]
                w1s = wv[i * 2 * ct + 2 * u + 1, :]
                for gi in range(d // _LANES):
                    sl = pl.ds(gi * _LANES, _LANES)
                    obb[u, sl] = rb[2 * u, sl] * w0 + rb[2 * u + 1, sl] * w1s
                return c2

            lax.fori_loop(0, ct, tok, 0)
            o[i] = pltpu.async_copy(
                obb, out_hbm.at[pl.ds(t_base + i * ct, ct)], osem[b])
        for j in range(max(0, nch - 2), nch):
            o[j].wait()

    return ck(ysorted, dest, wexp)


def kernel(x, gate_w, w1, w2, w3):
    b, s, d = x.shape
    e, f, _ = w1.shape
    s_tot = b * s
    k = 2
    tb = 64
    ch = 32
    pt = s_tot * k + e * tb
    nb = pt // tb
    n_flat = s_tot * k

    xf = x.reshape(s_tot, d)
    x3 = xf.reshape(s_tot, 8, 128)
    eidx, wts = _gate(xf, gate_w)

    eflat = eidx.reshape(n_flat)
    wflat = wts.reshape(n_flat)
    i_arange = jnp.arange(n_flat, dtype=jnp.int32)
    key = eflat * n_flat + i_arange
    skey = lax.sort(key)
    sorted_e = skey // n_flat
    perm = skey % n_flat
    boundary = jnp.concatenate(
        [jnp.ones((1,), jnp.int32),
         (sorted_e[1:] != sorted_e[:-1]).astype(jnp.int32)])
    seg_start = lax.cummax(jnp.where(boundary == 1, i_arange, 0))
    prev_start = jnp.concatenate([jnp.zeros((1,), jnp.int32),
                                  seg_start[:-1]])
    prev_len = i_arange - prev_start
    pad_amt = jnp.where((boundary == 1) & (i_arange > 0),
                        (-prev_len) % tb, 0)
    pos = i_arange + jnp.cumsum(pad_amt)
    _, dest = lax.sort((perm, pos), num_keys=1)
    blk = pos // tb
    in_blk = blk[None, :] == jnp.arange(nb, dtype=jnp.int32)[:, None]
    b2e = lax.cummax(
        jnp.max(jnp.where(in_blk, sorted_e[None, :], 0), axis=1))
    act = (jnp.arange(nb, dtype=jnp.int32) <= blk[-1]).astype(jnp.int32)
    tok2 = (perm // k).astype(jnp.int32).reshape(n_flat // ch, ch)
    pos2 = pos.reshape(n_flat // ch, ch)
    wexp = jnp.broadcast_to(wflat[:, None], (n_flat, _LANES))

    xsorted3 = _sc_dispatch(x3, tok2, pos2, pt)
    ysorted = _ffn(b2e, act, xsorted3, w1, w3, w2, tb)
    out = _sc_combine(ysorted, dest, wexp, s_tot)
    return out.reshape(b, s, d)


# speedup vs baseline: 4.8094x; 1.0332x over previous
import functools

import jax
import jax.numpy as jnp
from jax import lax
from jax.experimental import pallas as pl
from jax.experimental.pallas import tpu as pltpu
from jax.experimental.pallas import tpu_sc as plsc

_NC = 2
_NS = 16
_NW = _NC * _NS
_LANES = 16


def _gate_body(x_ref, gw_ref, eidx_ref, w_ref):
    x = x_ref[...]
    s = lax.dot_general(x, gw_ref[...], (((1,), (1,)), ((), ())),
                        preferred_element_type=jnp.float32)
    lane = lax.broadcasted_iota(jnp.int32, s.shape, 1)
    m1 = jnp.max(s, axis=1)
    a1 = jnp.argmax(s, axis=1).astype(jnp.int32)
    s2 = jnp.where(lane == a1[:, None], jnp.float32(-1e30), s)
    m2 = jnp.max(s2, axis=1)
    a2 = jnp.argmax(s2, axis=1).astype(jnp.int32)
    p1 = jax.nn.sigmoid(m1 - m2)
    eidx_ref[...] = jnp.stack([a1, a2], axis=1)
    w_ref[...] = jnp.stack([p1, 1.0 - p1], axis=1)


def _gate(xf, gate_w):
    s_tot, d = xf.shape
    e = gate_w.shape[0]
    bt = 256
    return pl.pallas_call(
        _gate_body,
        grid=(s_tot // bt,),
        in_specs=[
            pl.BlockSpec((bt, d), lambda b: (b, 0)),
            pl.BlockSpec((e, d), lambda b: (0, 0)),
        ],
        out_specs=[
            pl.BlockSpec((bt, 2), lambda b: (b, 0)),
            pl.BlockSpec((bt, 2), lambda b: (b, 0)),
        ],
        out_shape=[
            jax.ShapeDtypeStruct((s_tot, 2), jnp.int32),
            jax.ShapeDtypeStruct((s_tot, 2), jnp.float32),
        ],
    )(xf, gate_w)


def _sc_dispatch(x3, tok2, pos2, pt):
    nch_tot, ch = tok2.shape
    nch = nch_tot // _NW
    ring = 3
    mesh = plsc.VectorSubcoreMesh(core_axis_name="c", subcore_axis_name="s")

    @functools.partial(
        pl.kernel, mesh=mesh,
        out_type=jax.ShapeDtypeStruct((pt, 8, 128), jnp.float32),
        scratch_types=[
            pltpu.VMEM((nch, ch), jnp.int32),
            pltpu.VMEM((nch, ch), jnp.int32),
            pltpu.VMEM((ring, ch, 8, 128), jnp.float32),
            pltpu.SemaphoreType.DMA,
            pltpu.SemaphoreType.DMA,
            pltpu.SemaphoreType.DMA,
            pltpu.SemaphoreType.DMA,
            pltpu.SemaphoreType.DMA,
            pltpu.SemaphoreType.DMA,
        ],
    )
    def dk(x_hbm, tok_hbm, pos_hbm, out_hbm, tok_v, pos_v, rows_v,
           g0, g1, g2, o0, o1, o2):
        gsem = (g0, g1, g2)
        osem = (o0, o1, o2)
        wid = lax.axis_index("s") * _NC + lax.axis_index("c")
        pltpu.sync_copy(tok_hbm.at[pl.ds(wid * nch, nch)], tok_v)
        pltpu.sync_copy(pos_hbm.at[pl.ds(wid * nch, nch)], pos_v)

        def start_g(i):
            b = i % ring
            return pltpu.async_copy(
                x_hbm.at[tok_v.at[i]], rows_v.at[b], gsem[b])

        def start_o(i):
            b = i % ring
            return pltpu.async_copy(
                rows_v.at[b], out_hbm.at[pos_v.at[i]], osem[b])

        g = [None] * nch
        o = [None] * nch
        g[0] = start_g(0)
        for i in range(nch):
            if i + 1 < nch:
                if i + 1 >= ring:
                    o[i + 1 - ring].wait()
                g[i + 1] = start_g(i + 1)
            g[i].wait()
            o[i] = start_o(i)
        for j in range(max(0, nch - ring), nch):
            o[j].wait()

    return dk(x3, tok2, pos2)


def _ffn_body(b2e_ref, act_ref, xs_ref, w1_ref, w3_ref, w2_ref, out_ref):
    @pl.when(act_ref[pl.program_id(0)] != 0)
    def _():
        xs = xs_ref[...].reshape(xs_ref.shape[0], 1024)
        a = lax.dot_general(xs, w1_ref[0], (((1,), (1,)), ((), ())),
                            preferred_element_type=jnp.float32)
        c = lax.dot_general(xs, w3_ref[0], (((1,), (1,)), ((), ())),
                            preferred_element_type=jnp.float32)
        h = a * jax.nn.sigmoid(a) * c
        y = lax.dot_general(h, w2_ref[0], (((1,), (1,)), ((), ())),
                            preferred_element_type=jnp.float32)
        out_ref[...] = y


def _ffn(b2e, act, xsorted3, w1, w3, w2, tb):
    pt = xsorted3.shape[0]
    e, f, d = w1.shape
    nb = pt // tb
    def _xs_map(b, b2e_ref, act_ref):
        return (jnp.where(act_ref[b] == 1, b, nb - 1), 0, 0)

    def _out_map(b, b2e_ref, act_ref):
        return (jnp.where(act_ref[b] == 1, b, nb - 1), 0)

    grid_spec = pltpu.PrefetchScalarGridSpec(
        num_scalar_prefetch=2,
        grid=(nb,),
        in_specs=[
            pl.BlockSpec((tb, 8, 128), _xs_map),
            pl.BlockSpec((1, f, d),
                         lambda b, b2e_ref, act_ref: (b2e_ref[b], 0, 0)),
            pl.BlockSpec((1, f, d),
                         lambda b, b2e_ref, act_ref: (b2e_ref[b], 0, 0)),
            pl.BlockSpec((1, d, f),
                         lambda b, b2e_ref, act_ref: (b2e_ref[b], 0, 0)),
        ],
        out_specs=pl.BlockSpec((tb, d), _out_map),
    )
    return pl.pallas_call(
        _ffn_body,
        grid_spec=grid_spec,
        out_shape=jax.ShapeDtypeStruct((pt, d), jnp.float32),
        compiler_params=pltpu.CompilerParams(
            dimension_semantics=("arbitrary",)),
    )(b2e, act, xsorted3, w1, w3, w2)


def _sc_combine(ysorted, dest, wexp, s_tot):
    d = ysorted.shape[1]
    ct = 16
    tok_per_w = s_tot // _NW
    nch = tok_per_w // ct
    mesh = plsc.VectorSubcoreMesh(core_axis_name="c", subcore_axis_name="s")

    @functools.partial(
        pl.kernel, mesh=mesh,
        out_type=jax.ShapeDtypeStruct((s_tot, d), jnp.float32),
        scratch_types=[
            pltpu.VMEM((2 * tok_per_w,), jnp.int32),
            pltpu.VMEM((2 * tok_per_w, _LANES), jnp.float32),
            pltpu.VMEM((2, 2 * ct, d), jnp.float32),
            pltpu.VMEM((2, ct, d), jnp.float32),
            pltpu.SemaphoreType.DMA,
            pltpu.SemaphoreType.DMA,
            pltpu.SemaphoreType.DMA,
            pltpu.SemaphoreType.DMA,
        ],
    )
    def ck(y_hbm, dest_hbm, w_hbm, out_hbm, idx_v, wv, rows_v, ob_v,
           g0, g1, o0, o1):
        gsem = (g0, g1)
        osem = (o0, o1)
        wid = lax.axis_index("s") * _NC + lax.axis_index("c")
        t_base = wid * tok_per_w
        pltpu.sync_copy(dest_hbm.at[pl.ds(2 * t_base, 2 * tok_per_w)], idx_v)
        pltpu.sync_copy(w_hbm.at[pl.ds(2 * t_base, 2 * tok_per_w)], wv)

        def start_gather(i):
            b = i % 2
            return pltpu.async_copy(
                y_hbm.at[idx_v.at[pl.ds(i * 2 * ct, 2 * ct)]],
                rows_v.at[b], gsem[b])

        g = [None] * nch
        o = [None] * nch
        g[0] = start_gather(0)
        for i in range(nch):
            b = i % 2
            if i + 1 < nch:
                g[i + 1] = start_gather(i + 1)
            g[i].wait()
            if i >= 2:
                o[i - 2].wait()
            rb = rows_v.at[b]
            obb = ob_v.at[b]

            def tok(u, c2):
                w0 = wv[i * 2 * ct + 2 * u, :]
                w1s = wv[i * 2 * ct + 2 * u + 1, :]
                for gi in range(d // _LANES):
                    sl = pl.ds(gi * _LANES, _LANES)
                    obb[u, sl] = rb[2 * u, sl] * w0 + rb[2 * u + 1, sl] * w1s
                return c2

            lax.fori_loop(0, ct, tok, 0)
            o[i] = pltpu.async_copy(
                obb, out_hbm.at[pl.ds(t_base + i * ct, ct)], osem[b])
        for j in range(max(0, nch - 2), nch):
            o[j].wait()

    return ck(ysorted, dest, wexp)


def kernel(x, gate_w, w1, w2, w3):
    b, s, d = x.shape
    e, f, _ = w1.shape
    s_tot = b * s
    k = 2
    tb = 64
    ch = 32
    pt = s_tot * k + e * tb
    nb = pt // tb
    n_flat = s_tot * k

    xf = x.reshape(s_tot, d)
    x3 = xf.reshape(s_tot, 8, 128)
    eidx, wts = _gate(xf, gate_w)

    eflat = eidx.reshape(n_flat)
    wflat = wts.reshape(n_flat)
    i_arange = jnp.arange(n_flat, dtype=jnp.int32)
    key = eflat * n_flat + i_arange
    skey = lax.sort(key)
    sorted_e = skey // n_flat
    perm = skey % n_flat
    boundary = jnp.concatenate(
        [jnp.ones((1,), jnp.int32),
         (sorted_e[1:] != sorted_e[:-1]).astype(jnp.int32)])
    seg_start = lax.cummax(jnp.where(boundary == 1, i_arange, 0))
    prev_start = jnp.concatenate([jnp.zeros((1,), jnp.int32),
                                  seg_start[:-1]])
    prev_len = i_arange - prev_start
    pad_amt = jnp.where((boundary == 1) & (i_arange > 0),
                        (-prev_len) % tb, 0)
    pos = i_arange + jnp.cumsum(pad_amt)
    _, dest = lax.sort((perm, pos), num_keys=1)
    blk = pos // tb
    in_blk = blk[None, :] == jnp.arange(nb, dtype=jnp.int32)[:, None]
    b2e = lax.cummax(
        jnp.max(jnp.where(in_blk, sorted_e[None, :], 0), axis=1))
    act = (jnp.arange(nb, dtype=jnp.int32) <= blk[-1]).astype(jnp.int32)
    tok2 = (perm // k).astype(jnp.int32).reshape(n_flat // ch, ch)
    pos2 = pos.reshape(n_flat // ch, ch)
    wexp = jnp.broadcast_to(wflat[:, None], (n_flat, _LANES))

    xsorted3 = _sc_dispatch(x3, tok2, pos2, pt)
    ysorted = _ffn(b2e, act, xsorted3, w1, w3, w2, tb)
    out = _sc_combine(ysorted, dest, wexp, s_tot)
    return out.reshape(b, s, d)


# speedup vs baseline: 4.8139x; 1.0009x over previous
import functools

import jax
import jax.numpy as jnp
from jax import lax
from jax.experimental import pallas as pl
from jax.experimental.pallas import tpu as pltpu
from jax.experimental.pallas import tpu_sc as plsc

_NC = 2
_NS = 16
_NW = _NC * _NS
_LANES = 16


def _gate_body(x_ref, gw_ref, eidx_ref, w_ref):
    x = x_ref[...]
    s = lax.dot_general(x, gw_ref[...], (((1,), (1,)), ((), ())),
                        preferred_element_type=jnp.float32)
    lane = lax.broadcasted_iota(jnp.int32, s.shape, 1)
    m1 = jnp.max(s, axis=1)
    a1 = jnp.argmax(s, axis=1).astype(jnp.int32)
    s2 = jnp.where(lane == a1[:, None], jnp.float32(-1e30), s)
    m2 = jnp.max(s2, axis=1)
    a2 = jnp.argmax(s2, axis=1).astype(jnp.int32)
    p1 = jax.nn.sigmoid(m1 - m2)
    eidx_ref[...] = jnp.stack([a1, a2], axis=1)
    w_ref[...] = jnp.stack([p1, 1.0 - p1], axis=1)


def _gate(xf, gate_w):
    s_tot, d = xf.shape
    e = gate_w.shape[0]
    bt = 256
    return pl.pallas_call(
        _gate_body,
        grid=(s_tot // bt,),
        in_specs=[
            pl.BlockSpec((bt, d), lambda b: (b, 0)),
            pl.BlockSpec((e, d), lambda b: (0, 0)),
        ],
        out_specs=[
            pl.BlockSpec((bt, 2), lambda b: (b, 0)),
            pl.BlockSpec((bt, 2), lambda b: (b, 0)),
        ],
        out_shape=[
            jax.ShapeDtypeStruct((s_tot, 2), jnp.int32),
            jax.ShapeDtypeStruct((s_tot, 2), jnp.float32),
        ],
    )(xf, gate_w)


def _sc_dispatch(x3, tok2, pos2, pt):
    nch_tot, ch = tok2.shape
    nch = nch_tot // _NW
    ring = 3
    mesh = plsc.VectorSubcoreMesh(core_axis_name="c", subcore_axis_name="s")

    @functools.partial(
        pl.kernel, mesh=mesh,
        out_type=jax.ShapeDtypeStruct((pt, 8, 128), jnp.float32),
        scratch_types=[
            pltpu.VMEM((nch, ch), jnp.int32),
            pltpu.VMEM((nch, ch), jnp.int32),
            pltpu.VMEM((ring, ch, 8, 128), jnp.float32),
            pltpu.SemaphoreType.DMA,
            pltpu.SemaphoreType.DMA,
            pltpu.SemaphoreType.DMA,
            pltpu.SemaphoreType.DMA,
            pltpu.SemaphoreType.DMA,
            pltpu.SemaphoreType.DMA,
        ],
    )
    def dk(x_hbm, tok_hbm, pos_hbm, out_hbm, tok_v, pos_v, rows_v,
           g0, g1, g2, o0, o1, o2):
        gsem = (g0, g1, g2)
        osem = (o0, o1, o2)
        wid = lax.axis_index("s") * _NC + lax.axis_index("c")
        pltpu.sync_copy(tok_hbm.at[pl.ds(wid * nch, nch)], tok_v)
        pltpu.sync_copy(pos_hbm.at[pl.ds(wid * nch, nch)], pos_v)

        def start_g(i):
            b = i % ring
            return pltpu.async_copy(
                x_hbm.at[tok_v.at[i]], rows_v.at[b], gsem[b])

        def start_o(i):
            b = i % ring
            return pltpu.async_copy(
                rows_v.at[b], out_hbm.at[pos_v.at[i]], osem[b])

        g = [None] * nch
        o = [None] * nch
        g[0] = start_g(0)
        for i in range(nch):
            if i + 1 < nch:
                if i + 1 >= ring:
                    o[i + 1 - ring].wait()
                g[i + 1] = start_g(i + 1)
            g[i].wait()
            o[i] = start_o(i)
        for j in range(max(0, nch - ring), nch):
            o[j].wait()

    return dk(x3, tok2, pos2)


def _ffn_body(b2e_ref, act_ref, xs_ref, w1_ref, w3_ref, w2_ref, out_ref):
    @pl.when(act_ref[pl.program_id(0)] != 0)
    def _():
        xs = xs_ref[...].reshape(xs_ref.shape[0], 1024)
        a = lax.dot_general(xs, w1_ref[0], (((1,), (1,)), ((), ())),
                            preferred_element_type=jnp.float32)
        c = lax.dot_general(xs, w3_ref[0], (((1,), (1,)), ((), ())),
                            preferred_element_type=jnp.float32)
        h = a * jax.nn.sigmoid(a) * c
        y = lax.dot_general(h, w2_ref[0], (((1,), (1,)), ((), ())),
                            preferred_element_type=jnp.float32)
        out_ref[...] = y.reshape(out_ref.shape)


def _ffn(b2e, act, xsorted3, w1, w3, w2, tb):
    pt = xsorted3.shape[0]
    e, f, d = w1.shape
    nb = pt // tb
    def _xs_map(b, b2e_ref, act_ref):
        return (jnp.where(act_ref[b] == 1, b, nb - 1), 0, 0)

    def _out_map(b, b2e_ref, act_ref):
        return (jnp.where(act_ref[b] == 1, b, nb - 1), 0, 0)

    grid_spec = pltpu.PrefetchScalarGridSpec(
        num_scalar_prefetch=2,
        grid=(nb,),
        in_specs=[
            pl.BlockSpec((tb, 8, 128), _xs_map),
            pl.BlockSpec((1, f, d),
                         lambda b, b2e_ref, act_ref: (b2e_ref[b], 0, 0)),
            pl.BlockSpec((1, f, d),
                         lambda b, b2e_ref, act_ref: (b2e_ref[b], 0, 0)),
            pl.BlockSpec((1, d, f),
                         lambda b, b2e_ref, act_ref: (b2e_ref[b], 0, 0)),
        ],
        out_specs=pl.BlockSpec((tb, 8, 128), _out_map),
    )
    return pl.pallas_call(
        _ffn_body,
        grid_spec=grid_spec,
        out_shape=jax.ShapeDtypeStruct((pt, 8, 128), jnp.float32),
        compiler_params=pltpu.CompilerParams(
            dimension_semantics=("arbitrary",)),
    )(b2e, act, xsorted3, w1, w3, w2)


def _sc_combine(ysorted3, dest, wexp, s_tot):
    d = 8 * 128
    ct = 16
    tok_per_w = s_tot // _NW
    nch = tok_per_w // ct
    mesh = plsc.VectorSubcoreMesh(core_axis_name="c", subcore_axis_name="s")

    @functools.partial(
        pl.kernel, mesh=mesh,
        out_type=jax.ShapeDtypeStruct((s_tot, d), jnp.float32),
        scratch_types=[
            pltpu.VMEM((2 * tok_per_w,), jnp.int32),
            pltpu.VMEM((2 * tok_per_w, _LANES), jnp.float32),
            pltpu.VMEM((2, 2 * ct, 8, 128), jnp.float32),
            pltpu.VMEM((2, ct, d), jnp.float32),
            pltpu.SemaphoreType.DMA,
            pltpu.SemaphoreType.DMA,
            pltpu.SemaphoreType.DMA,
            pltpu.SemaphoreType.DMA,
        ],
    )
    def ck(y_hbm, dest_hbm, w_hbm, out_hbm, idx_v, wv, rows_v, ob_v,
           g0, g1, o0, o1):
        gsem = (g0, g1)
        osem = (o0, o1)
        wid = lax.axis_index("s") * _NC + lax.axis_index("c")
        t_base = wid * tok_per_w
        pltpu.sync_copy(dest_hbm.at[pl.ds(2 * t_base, 2 * tok_per_w)], idx_v)
        pltpu.sync_copy(w_hbm.at[pl.ds(2 * t_base, 2 * tok_per_w)], wv)

        def start_gather(i):
            b = i % 2
            return pltpu.async_copy(
                y_hbm.at[idx_v.at[pl.ds(i * 2 * ct, 2 * ct)]],
                rows_v.at[b], gsem[b])

        g = [None] * nch
        o = [None] * nch
        g[0] = start_gather(0)
        for i in range(nch):
            b = i % 2
            if i + 1 < nch:
                g[i + 1] = start_gather(i + 1)
            g[i].wait()
            if i >= 2:
                o[i - 2].wait()
            rb = rows_v.at[b]
            obb = ob_v.at[b]

            def tok(u, c2):
                w0 = wv[i * 2 * ct + 2 * u, :]
                w1s = wv[i * 2 * ct + 2 * u + 1, :]
                for si in range(8):
                    for gi in range(128 // _LANES):
                        sl = pl.ds(gi * _LANES, _LANES)
                        osl = pl.ds(si * 128 + gi * _LANES, _LANES)
                        obb[u, osl] = (rb[2 * u, si, sl] * w0 +
                                       rb[2 * u + 1, si, sl] * w1s)
                return c2

            lax.fori_loop(0, ct, tok, 0)
            o[i] = pltpu.async_copy(
                obb, out_hbm.at[pl.ds(t_base + i * ct, ct)], osem[b])
        for j in range(max(0, nch - 2), nch):
            o[j].wait()

    return ck(ysorted3, dest, wexp)


def kernel(x, gate_w, w1, w2, w3):
    b, s, d = x.shape
    e, f, _ = w1.shape
    s_tot = b * s
    k = 2
    tb = 64
    ch = 32
    pt = s_tot * k + e * tb
    nb = pt // tb
    n_flat = s_tot * k

    xf = x.reshape(s_tot, d)
    x3 = xf.reshape(s_tot, 8, 128)
    eidx, wts = _gate(xf, gate_w)

    eflat = eidx.reshape(n_flat)
    wflat = wts.reshape(n_flat)
    i_arange = jnp.arange(n_flat, dtype=jnp.int32)
    key = eflat * n_flat + i_arange
    skey = lax.sort(key)
    sorted_e = skey // n_flat
    perm = skey % n_flat
    boundary = jnp.concatenate(
        [jnp.ones((1,), jnp.int32),
         (sorted_e[1:] != sorted_e[:-1]).astype(jnp.int32)])
    seg_start = lax.cummax(jnp.where(boundary == 1, i_arange, 0))
    prev_start = jnp.concatenate([jnp.zeros((1,), jnp.int32),
                                  seg_start[:-1]])
    prev_len = i_arange - prev_start
    pad_amt = jnp.where((boundary == 1) & (i_arange > 0),
                        (-prev_len) % tb, 0)
    pos = i_arange + jnp.cumsum(pad_amt)
    _, dest = lax.sort((perm, pos), num_keys=1)
    blk = pos // tb
    in_blk = blk[None, :] == jnp.arange(nb, dtype=jnp.int32)[:, None]
    b2e = lax.cummax(
        jnp.max(jnp.where(in_blk, sorted_e[None, :], 0), axis=1))
    act = (jnp.arange(nb, dtype=jnp.int32) <= blk[-1]).astype(jnp.int32)
    tok2 = (perm // k).astype(jnp.int32).reshape(n_flat // ch, ch)
    pos2 = pos.reshape(n_flat // ch, ch)
    wexp = jnp.broadcast_to(wflat[:, None], (n_flat, _LANES))

    xsorted3 = _sc_dispatch(x3, tok2, pos2, pt)
    ysorted = _ffn(b2e, act, xsorted3, w1, w3, w2, tb)
    out = _sc_combine(ysorted, dest, wexp, s_tot)
    return out.reshape(b, s, d)
